# Initial kernel scaffold; baseline (speedup 1.0000x reference)
import functools

import jax
import jax.numpy as jnp
from jax import lax
from jax.experimental import pallas as pl
from jax.experimental.pallas import tpu as pltpu
from jax.experimental.pallas import tpu_sc as plsc

DIM = 32
LANE = 128
NW = 32
B = 16384 * 50
B_PER_W = B // NW
CHUNK = 1280
N_CHUNK = B_PER_W // CHUNK
IDX_ROWS = CHUNK // LANE

_mesh = plsc.VectorSubcoreMesh(core_axis_name="c", subcore_axis_name="s")


@functools.partial(
    pl.kernel,
    mesh=_mesh,
    out_type=jax.ShapeDtypeStruct((B, DIM), jnp.float32),
    scratch_types=[
        pltpu.VMEM((IDX_ROWS, LANE), jnp.int32),
        pltpu.VMEM((CHUNK, DIM), jnp.float32),
        pltpu.SemaphoreType.DMA,
    ],
)
def _gather(x_hbm, table_hbm, out_hbm, idx_v, rows_v, sem):
    wid = lax.axis_index("s") * 2 + lax.axis_index("c")
    row0 = wid * (B_PER_W // LANE)

    def step(g, carry):
        r = row0 + g * IDX_ROWS
        pltpu.sync_copy(x_hbm.at[pl.ds(r, IDX_ROWS)], idx_v)
        copies = []
        for j in range(IDX_ROWS):
            copies.append(
                pltpu.async_copy(
                    table_hbm.at[idx_v.at[j]],
                    rows_v.at[pl.ds(j * LANE, LANE)],
                    sem,
                )
            )
        for c in copies:
            c.wait()
        pltpu.sync_copy(rows_v, out_hbm.at[pl.ds(r * LANE, CHUNK)])
        return carry

    lax.fori_loop(0, N_CHUNK, step, 0)


def kernel(x, table):
    flat = x.reshape(-1).astype(jnp.int32).reshape(B // LANE, LANE)
    out = _gather(flat, table)
    return out.reshape(x.shape + (DIM,))


# speedup vs baseline: 1.0947x; 1.0947x over previous
import functools

import jax
import jax.numpy as jnp
from jax import lax
from jax.experimental import pallas as pl
from jax.experimental.pallas import tpu as pltpu
from jax.experimental.pallas import tpu_sc as plsc

DIM = 32
LANE = 128
NW = 32
B = 16384 * 50
B_PER_W = B // NW
CHUNK = 1024
N_CHUNK = B_PER_W // CHUNK
IDX_ROWS = CHUNK // LANE

_mesh = plsc.VectorSubcoreMesh(core_axis_name="c", subcore_axis_name="s")


@functools.partial(
    pl.kernel,
    mesh=_mesh,
    out_type=jax.ShapeDtypeStruct((B, DIM), jnp.float32),
    scratch_types=[
        pltpu.VMEM((IDX_ROWS, LANE), jnp.int32),
        pltpu.VMEM((CHUNK, DIM), jnp.float32),
        pltpu.SemaphoreType.DMA,
    ],
    compiler_params=pltpu.CompilerParams(use_tc_tiling_on_sc=False),
)
def _gather(x_hbm, table_hbm, out_hbm, idx_v, rows_v, sem):
    wid = lax.axis_index("s") * 2 + lax.axis_index("c")
    row0 = wid * (B_PER_W // LANE)

    def step(g, carry):
        r = row0 + g * IDX_ROWS
        pltpu.sync_copy(x_hbm.at[pl.ds(r, IDX_ROWS)], idx_v)
        copies = []
        for j in range(IDX_ROWS):
            copies.append(
                pltpu.async_copy(
                    table_hbm.at[idx_v.at[j]],
                    rows_v.at[pl.ds(j * LANE, LANE)],
                    sem,
                )
            )
        for c in copies:
            c.wait()
        pltpu.sync_copy(rows_v, out_hbm.at[pl.ds(r * LANE, CHUNK)])
        return carry

    lax.fori_loop(0, N_CHUNK, step, 0)


def kernel(x, table):
    flat = x.reshape(-1).astype(jnp.int32).reshape(B // LANE, LANE)
    out = _gather(flat, table)
    return out.reshape(x.shape + (DIM,))
